# Initial kernel scaffold; baseline (speedup 1.0000x reference)
import jax
import jax.numpy as jnp
from jax.experimental import pallas as pl


def kernel(x, edge_index, W1, b1, W2, b2):
    raise NotImplementedError("write your pallas kernel here")


# speedup vs baseline: 27.7438x; 27.7438x over previous
import functools

import jax
import jax.numpy as jnp
from jax import lax
from jax.experimental import pallas as pl
from jax.experimental.pallas import tpu as pltpu
from jax.experimental.pallas import tpu_sc as plsc

N = 10000
E = 320000
D_IN = 128
D_HID = 256
D_OUT = 128

NC = 2
NS = 16

KE = 125
GC = 8
L1_GROUPS = E // NS // (GC * KE)
L2_GROUPS = E // (NC * NS) // (GC * KE)
DEG_CHUNKS = E // (NC * NS) // KE


def _zero_fill(buf, nrows, ncols):
    z16 = jnp.zeros((16,), jnp.float32)
    per_row = ncols // 16

    def body(t, carry):
        i = t // per_row
        k = t - i * per_row
        buf[i, pl.ds(k * 16, 16)] = z16
        return carry

    lax.fori_loop(0, nrows * per_row, body, 0)


def _zero_acc_slice(acc, zbuf, s, width):
    _zero_fill(zbuf, KE, width)
    for q in range(5):
        pltpu.sync_copy(zbuf, acc.at[pl.ds(s * 625 + q * KE, KE)])


def _agg_pipeline(tbl, eidx, lead, acc, ib, rb, semi, semg, ngroups):
    nchunks = ngroups * GC

    pltpu.sync_copy(eidx.at[lead, 0], ib[0])
    pltpu.async_copy(tbl.at[ib[0].at[0]], rb[0], semg[0])

    def group_pair(gp, carry):
        for half in (0, 1):
            g = 2 * gp + half
            ibc, ibn = ib[half], ib[1 - half]
            for k in range(GC):
                j = g * GC + k
                p = (GC * half + k) % 2
                rc, rn = rb[p], rb[1 - p]
                sc, sn = semg[p], semg[1 - p]
                if k == 0:
                    @pl.when(g + 1 < ngroups)
                    def _():
                        pltpu.async_copy(eidx.at[lead, g + 1], ibn, semi)
                if k < GC - 1:
                    pltpu.async_copy(tbl.at[ibc.at[k + 1]], rn, sn)
                else:
                    @pl.when(j + 1 < nchunks)
                    def _():
                        pltpu.make_async_copy(eidx.at[lead, g + 1], ibn, semi).wait()
                        pltpu.async_copy(tbl.at[ibn.at[0]], rn, sn)
                pltpu.make_async_copy(tbl.at[ibc.at[k]], rc, sc).wait()
                pltpu.sync_copy(rc, acc.at[ibc.at[GC + k]], add=True)
        return carry

    lax.fori_loop(0, ngroups // 2, group_pair, 0)


def _flush(acc, out_h, c, s, core):
    @pl.when((c == core) & (s < 10))
    def _():
        sl = pl.ds(s * 1000, 1000)
        pltpu.sync_copy(acc.at[sl], out_h.at[sl])


def _sc_mesh():
    return plsc.VectorSubcoreMesh(core_axis_name="c", subcore_axis_name="s",
                                  num_cores=NC, num_subcores=NS)


@functools.cache
def _make_deg_kernel():
    return pl.kernel(
        _deg_body,
        out_type=(
            jax.ShapeDtypeStruct((N, 16), jnp.float32),
            jax.ShapeDtypeStruct((N, 16), jnp.float32),
        ),
        mesh=_sc_mesh(),
        scratch_types=(
            pltpu.VMEM_SHARED((N, 16), jnp.float32),
            pltpu.VMEM((DEG_CHUNKS, KE), jnp.int32),
            pltpu.VMEM((KE, 16), jnp.float32),
            pltpu.VMEM((KE, 16), jnp.float32),
        ),
    )


def _deg_body(dst_h, dega_h, degb_h, acc, dst_v, ones_v, zbuf):
    c = lax.axis_index("c")
    s = lax.axis_index("s")
    w = c * NS + s

    _zero_fill(zbuf, KE, 16)
    for q in range(5):
        pltpu.sync_copy(zbuf, acc.at[pl.ds(s * 625 + q * KE, KE)])

    one16 = jnp.ones((16,), jnp.float32)

    def fill_ones(i, carry):
        ones_v[i] = one16
        return carry

    lax.fori_loop(0, KE, fill_ones, 0)

    pltpu.sync_copy(dst_h.at[w], dst_v)
    plsc.subcore_barrier()

    def body(j, carry):
        pltpu.sync_copy(ones_v, acc.at[dst_v.at[j]], add=True)
        return carry

    lax.fori_loop(0, DEG_CHUNKS, body, 0)
    plsc.subcore_barrier()

    _flush(acc, dega_h, c, s, 0)
    _flush(acc, degb_h, c, s, 1)


@functools.cache
def _make_agg1_kernel():
    return pl.kernel(
        _agg1_body,
        out_type=(
            jax.ShapeDtypeStruct((N, 128), jnp.float32),
            jax.ShapeDtypeStruct((N, 128), jnp.float32),
        ),
        mesh=_sc_mesh(),
        scratch_types=(
            pltpu.VMEM_SHARED((N, 128), jnp.float32),
            pltpu.VMEM((2 * GC, KE), jnp.int32),
            pltpu.VMEM((2 * GC, KE), jnp.int32),
            pltpu.VMEM((KE, 128), jnp.float32),
            pltpu.VMEM((KE, 128), jnp.float32),
            pltpu.SemaphoreType.DMA,
            pltpu.SemaphoreType.DMA,
            pltpu.SemaphoreType.DMA,
        ),
    )


def _agg1_body(ga_h, gb_h, eidx_h, outa_h, outb_h,
               acc, ib0, ib1, rb0, rb1, semi, semg0, semg1):
    c = lax.axis_index("c")
    s = lax.axis_index("s")

    _zero_acc_slice(acc, rb0, s, 128)
    plsc.subcore_barrier()

    @pl.when(c == 0)
    def _():
        _agg_pipeline(ga_h, eidx_h, s, acc, (ib0, ib1), (rb0, rb1),
                      semi, (semg0, semg1), L1_GROUPS)

    @pl.when(c == 1)
    def _():
        _agg_pipeline(gb_h, eidx_h, s, acc, (ib0, ib1), (rb0, rb1),
                      semi, (semg0, semg1), L1_GROUPS)

    plsc.subcore_barrier()
    _flush(acc, outa_h, c, s, 0)
    _flush(acc, outb_h, c, s, 1)


@functools.cache
def _make_agg2_kernel():
    return pl.kernel(
        _agg2_body,
        out_type=(
            jax.ShapeDtypeStruct((N, 128), jnp.float32),
            jax.ShapeDtypeStruct((N, 128), jnp.float32),
        ),
        mesh=_sc_mesh(),
        scratch_types=(
            pltpu.VMEM_SHARED((N, 128), jnp.float32),
            pltpu.VMEM((2 * GC, KE), jnp.int32),
            pltpu.VMEM((2 * GC, KE), jnp.int32),
            pltpu.VMEM((KE, 128), jnp.float32),
            pltpu.VMEM((KE, 128), jnp.float32),
            pltpu.SemaphoreType.DMA,
            pltpu.SemaphoreType.DMA,
            pltpu.SemaphoreType.DMA,
        ),
    )


def _agg2_body(g_h, eidx_h, outa_h, outb_h,
               acc, ib0, ib1, rb0, rb1, semi, semg0, semg1):
    c = lax.axis_index("c")
    s = lax.axis_index("s")
    w = c * NS + s

    _zero_acc_slice(acc, rb0, s, 128)
    plsc.subcore_barrier()

    _agg_pipeline(g_h, eidx_h, w, acc, (ib0, ib1), (rb0, rb1),
                  semi, (semg0, semg1), L2_GROUPS)

    plsc.subcore_barrier()
    _flush(acc, outa_h, c, s, 0)
    _flush(acc, outb_h, c, s, 1)


_BR = 1000
_GRID = N // _BR


def _dis(dega_ref, degb_ref):
    d = dega_ref[:, 0] + degb_ref[:, 0] + 1.0
    return lax.rsqrt(d)


def _tc1_body(x_ref, w1_ref, dega_ref, degb_ref, ga_ref, gb_ref):
    h = lax.dot_general(x_ref[...], w1_ref[...], (((1,), (0,)), ((), ())),
                        preferred_element_type=jnp.float32)
    g = h * _dis(dega_ref, degb_ref)[:, None]
    ga_ref[...] = g[:, :128]
    gb_ref[...] = g[:, 128:]


def _tc1(x, W1, dega, degb):
    return pl.pallas_call(
        _tc1_body,
        grid=(_GRID,),
        in_specs=[
            pl.BlockSpec((_BR, D_IN), lambda i: (i, 0)),
            pl.BlockSpec((D_IN, D_HID), lambda i: (0, 0)),
            pl.BlockSpec((_BR, 16), lambda i: (i, 0)),
            pl.BlockSpec((_BR, 16), lambda i: (i, 0)),
        ],
        out_specs=[
            pl.BlockSpec((_BR, 128), lambda i: (i, 0)),
            pl.BlockSpec((_BR, 128), lambda i: (i, 0)),
        ],
        out_shape=[
            jax.ShapeDtypeStruct((N, 128), jnp.float32),
            jax.ShapeDtypeStruct((N, 128), jnp.float32),
        ],
    )(x, W1, dega, degb)


def _tc2_body(agga_ref, aggb_ref, ga_ref, gb_ref, dega_ref, degb_ref,
              b1_ref, w2_ref, g2_ref):
    dis = _dis(dega_ref, degb_ref)[:, None]
    z0 = dis * (agga_ref[...] + ga_ref[...]) + b1_ref[0, :128][None, :]
    z1 = dis * (aggb_ref[...] + gb_ref[...]) + b1_ref[0, 128:][None, :]
    r0 = jnp.maximum(z0, 0.0)
    r1 = jnp.maximum(z1, 0.0)
    h2 = (lax.dot_general(r0, w2_ref[:128, :], (((1,), (0,)), ((), ())),
                          preferred_element_type=jnp.float32)
          + lax.dot_general(r1, w2_ref[128:, :], (((1,), (0,)), ((), ())),
                            preferred_element_type=jnp.float32))
    g2_ref[...] = h2 * dis


def _tc2(agg1a, agg1b, g1a, g1b, dega, degb, b1, W2):
    return pl.pallas_call(
        _tc2_body,
        grid=(_GRID,),
        in_specs=[
            pl.BlockSpec((_BR, 128), lambda i: (i, 0)),
            pl.BlockSpec((_BR, 128), lambda i: (i, 0)),
            pl.BlockSpec((_BR, 128), lambda i: (i, 0)),
            pl.BlockSpec((_BR, 128), lambda i: (i, 0)),
            pl.BlockSpec((_BR, 16), lambda i: (i, 0)),
            pl.BlockSpec((_BR, 16), lambda i: (i, 0)),
            pl.BlockSpec((1, D_HID), lambda i: (0, 0)),
            pl.BlockSpec((D_HID, D_OUT), lambda i: (0, 0)),
        ],
        out_specs=pl.BlockSpec((_BR, D_OUT), lambda i: (i, 0)),
        out_shape=jax.ShapeDtypeStruct((N, D_OUT), jnp.float32),
    )(agg1a, agg1b, g1a, g1b, dega, degb, b1, W2)


def _tc3_body(agga_ref, aggb_ref, g2_ref, dega_ref, degb_ref, b2_ref, out_ref):
    dis = _dis(dega_ref, degb_ref)[:, None]
    z = dis * (agga_ref[...] + aggb_ref[...] + g2_ref[...]) + b2_ref[0][None, :]
    out_ref[...] = jnp.maximum(z, 0.0)


def _tc3(agg2a, agg2b, g2, dega, degb, b2):
    return pl.pallas_call(
        _tc3_body,
        grid=(_GRID,),
        in_specs=[
            pl.BlockSpec((_BR, 128), lambda i: (i, 0)),
            pl.BlockSpec((_BR, 128), lambda i: (i, 0)),
            pl.BlockSpec((_BR, 128), lambda i: (i, 0)),
            pl.BlockSpec((_BR, 16), lambda i: (i, 0)),
            pl.BlockSpec((_BR, 16), lambda i: (i, 0)),
            pl.BlockSpec((1, D_OUT), lambda i: (0, 0)),
        ],
        out_specs=pl.BlockSpec((_BR, D_OUT), lambda i: (i, 0)),
        out_shape=jax.ShapeDtypeStruct((N, D_OUT), jnp.float32),
    )(agg2a, agg2b, g2, dega, degb, b2)


def _pack_eidx(src, dst, lead, groups):
    s4 = src.reshape(lead, groups, GC, KE)
    d4 = dst.reshape(lead, groups, GC, KE)
    return jnp.concatenate([s4, d4], axis=2)


def kernel(x, edge_index, W1, b1, W2, b2):
    ei = edge_index.astype(jnp.int32)
    src, dst = ei[0], ei[1]
    eidx1 = _pack_eidx(src, dst, NS, L1_GROUPS)
    eidx2 = _pack_eidx(src, dst, NC * NS, L2_GROUPS)
    dstD = dst.reshape(NC * NS, DEG_CHUNKS, KE)

    dega, degb = _make_deg_kernel()(dstD)
    g1a, g1b = _tc1(x, W1, dega, degb)
    agg1a, agg1b = _make_agg1_kernel()(g1a, g1b, eidx1)
    g2 = _tc2(agg1a, agg1b, g1a, g1b, dega, degb, b1.reshape(1, D_HID), W2)
    agg2a, agg2b = _make_agg2_kernel()(g2, eidx2)
    return _tc3(agg2a, agg2b, g2, dega, degb, b2.reshape(1, D_OUT))


# speedup vs baseline: 28.0905x; 1.0125x over previous
import functools

import jax
import jax.numpy as jnp
from jax import lax
from jax.experimental import pallas as pl
from jax.experimental.pallas import tpu as pltpu
from jax.experimental.pallas import tpu_sc as plsc

N = 10000
E = 320000
D_IN = 128
D_HID = 256
D_OUT = 128

NC = 2
NS = 16

KE = 125
GC = 8
L1_GROUPS = E // NS // (GC * KE)
L2_GROUPS = E // (NC * NS) // (GC * KE)
DEG_CHUNKS = E // (NC * NS) // KE


def _zero_fill(buf, nrows, ncols):
    z16 = jnp.zeros((16,), jnp.float32)
    per_row = ncols // 16

    def body(t, carry):
        i = t // per_row
        k = t - i * per_row
        buf[i, pl.ds(k * 16, 16)] = z16
        return carry

    lax.fori_loop(0, nrows * per_row, body, 0)


def _zero_acc_slice(acc, zbuf, s, width):
    _zero_fill(zbuf, KE, width)
    for q in range(5):
        pltpu.sync_copy(zbuf, acc.at[pl.ds(s * 625 + q * KE, KE)])


def _agg_pipeline(tbl, eidx, lead, acc, ib, rb, semi, semg, ngroups):
    nchunks = ngroups * GC

    pltpu.sync_copy(eidx.at[lead, 0], ib[0])
    pltpu.async_copy(tbl.at[ib[0].at[0]], rb[0], semg[0])

    def group_pair(gp, carry):
        for half in (0, 1):
            g = 2 * gp + half
            ibc, ibn = ib[half], ib[1 - half]
            for k in range(GC):
                j = g * GC + k
                p = (GC * half + k) % 2
                rc, rn = rb[p], rb[1 - p]
                sc, sn = semg[p], semg[1 - p]
                if k == 0:
                    @pl.when(g + 1 < ngroups)
                    def _():
                        pltpu.async_copy(eidx.at[lead, g + 1], ibn, semi)
                if k < GC - 1:
                    pltpu.async_copy(tbl.at[ibc.at[k + 1]], rn, sn)
                else:
                    @pl.when(j + 1 < nchunks)
                    def _():
                        pltpu.make_async_copy(eidx.at[lead, g + 1], ibn, semi).wait()
                        pltpu.async_copy(tbl.at[ibn.at[0]], rn, sn)
                pltpu.make_async_copy(tbl.at[ibc.at[k]], rc, sc).wait()
                pltpu.sync_copy(rc, acc.at[ibc.at[GC + k]], add=True)
        return carry

    lax.fori_loop(0, ngroups // 2, group_pair, 0)


def _flush(acc, out_h, c, s, core):
    @pl.when((c == core) & (s < 10))
    def _():
        sl = pl.ds(s * 1000, 1000)
        pltpu.sync_copy(acc.at[sl], out_h.at[sl])


def _sc_mesh():
    return plsc.VectorSubcoreMesh(core_axis_name="c", subcore_axis_name="s",
                                  num_cores=NC, num_subcores=NS)


@functools.cache
def _make_deg_kernel():
    return pl.kernel(
        _deg_body,
        out_type=(
            jax.ShapeDtypeStruct((N, 16), jnp.float32),
            jax.ShapeDtypeStruct((N, 16), jnp.float32),
        ),
        mesh=_sc_mesh(),
        scratch_types=(
            pltpu.VMEM_SHARED((N, 16), jnp.float32),
            pltpu.VMEM((DEG_CHUNKS, KE), jnp.int32),
            pltpu.VMEM((KE, 16), jnp.float32),
            pltpu.VMEM((KE, 16), jnp.float32),
            pltpu.SemaphoreType.DMA,
        ),
    )


def _deg_body(dst_h, dega_h, degb_h, acc, dst_v, ones_v, zbuf, semd):
    c = lax.axis_index("c")
    s = lax.axis_index("s")
    w = c * NS + s

    _zero_fill(zbuf, KE, 16)
    for q in range(5):
        pltpu.sync_copy(zbuf, acc.at[pl.ds(s * 625 + q * KE, KE)])

    one16 = jnp.ones((16,), jnp.float32)

    def fill_ones(i, carry):
        ones_v[i] = one16
        return carry

    lax.fori_loop(0, KE, fill_ones, 0)

    pltpu.sync_copy(dst_h.at[w], dst_v)
    plsc.subcore_barrier()

    def body(b, carry):
        for k in range(8):
            pltpu.async_copy(ones_v, acc.at[dst_v.at[8 * b + k]], semd, add=True)
        for k in range(8):
            pltpu.make_async_copy(ones_v, acc.at[dst_v.at[8 * b + k]], semd).wait()
        return carry

    lax.fori_loop(0, DEG_CHUNKS // 8, body, 0)
    plsc.subcore_barrier()

    _flush(acc, dega_h, c, s, 0)
    _flush(acc, degb_h, c, s, 1)


@functools.cache
def _make_agg1_kernel():
    return pl.kernel(
        _agg1_body,
        out_type=(
            jax.ShapeDtypeStruct((N, 128), jnp.float32),
            jax.ShapeDtypeStruct((N, 128), jnp.float32),
        ),
        mesh=_sc_mesh(),
        scratch_types=(
            pltpu.VMEM_SHARED((N, 128), jnp.float32),
            pltpu.VMEM((2 * GC, KE), jnp.int32),
            pltpu.VMEM((2 * GC, KE), jnp.int32),
            pltpu.VMEM((KE, 128), jnp.float32),
            pltpu.VMEM((KE, 128), jnp.float32),
            pltpu.SemaphoreType.DMA,
            pltpu.SemaphoreType.DMA,
            pltpu.SemaphoreType.DMA,
        ),
    )


def _agg1_body(ga_h, gb_h, eidx_h, outa_h, outb_h,
               acc, ib0, ib1, rb0, rb1, semi, semg0, semg1):
    c = lax.axis_index("c")
    s = lax.axis_index("s")

    _zero_acc_slice(acc, rb0, s, 128)
    plsc.subcore_barrier()

    @pl.when(c == 0)
    def _():
        _agg_pipeline(ga_h, eidx_h, s, acc, (ib0, ib1), (rb0, rb1),
                      semi, (semg0, semg1), L1_GROUPS)

    @pl.when(c == 1)
    def _():
        _agg_pipeline(gb_h, eidx_h, s, acc, (ib0, ib1), (rb0, rb1),
                      semi, (semg0, semg1), L1_GROUPS)

    plsc.subcore_barrier()
    _flush(acc, outa_h, c, s, 0)
    _flush(acc, outb_h, c, s, 1)


@functools.cache
def _make_agg2_kernel():
    return pl.kernel(
        _agg2_body,
        out_type=(
            jax.ShapeDtypeStruct((N, 128), jnp.float32),
            jax.ShapeDtypeStruct((N, 128), jnp.float32),
        ),
        mesh=_sc_mesh(),
        scratch_types=(
            pltpu.VMEM_SHARED((N, 128), jnp.float32),
            pltpu.VMEM((2 * GC, KE), jnp.int32),
            pltpu.VMEM((2 * GC, KE), jnp.int32),
            pltpu.VMEM((KE, 128), jnp.float32),
            pltpu.VMEM((KE, 128), jnp.float32),
            pltpu.SemaphoreType.DMA,
            pltpu.SemaphoreType.DMA,
            pltpu.SemaphoreType.DMA,
        ),
    )


def _agg2_body(g_h, eidx_h, outa_h, outb_h,
               acc, ib0, ib1, rb0, rb1, semi, semg0, semg1):
    c = lax.axis_index("c")
    s = lax.axis_index("s")
    w = c * NS + s

    _zero_acc_slice(acc, rb0, s, 128)
    plsc.subcore_barrier()

    _agg_pipeline(g_h, eidx_h, w, acc, (ib0, ib1), (rb0, rb1),
                  semi, (semg0, semg1), L2_GROUPS)

    plsc.subcore_barrier()
    _flush(acc, outa_h, c, s, 0)
    _flush(acc, outb_h, c, s, 1)


_BR = 1000
_GRID = N // _BR


def _dis(dega_ref, degb_ref):
    d = dega_ref[:, 0] + degb_ref[:, 0] + 1.0
    return lax.rsqrt(d)


def _tc0_body(x_ref, w1_ref, ha_ref, hb_ref):
    h = lax.dot_general(x_ref[...], w1_ref[...], (((1,), (0,)), ((), ())),
                        preferred_element_type=jnp.float32)
    ha_ref[...] = h[:, :128]
    hb_ref[...] = h[:, 128:]


def _tc0(x, W1):
    return pl.pallas_call(
        _tc0_body,
        grid=(_GRID,),
        in_specs=[
            pl.BlockSpec((_BR, D_IN), lambda i: (i, 0)),
            pl.BlockSpec((D_IN, D_HID), lambda i: (0, 0)),
        ],
        out_specs=[
            pl.BlockSpec((_BR, 128), lambda i: (i, 0)),
            pl.BlockSpec((_BR, 128), lambda i: (i, 0)),
        ],
        out_shape=[
            jax.ShapeDtypeStruct((N, 128), jnp.float32),
            jax.ShapeDtypeStruct((N, 128), jnp.float32),
        ],
    )(x, W1)


def _tc1_body(ha_ref, hb_ref, dega_ref, degb_ref, ga_ref, gb_ref):
    dis = _dis(dega_ref, degb_ref)[:, None]
    ga_ref[...] = ha_ref[...] * dis
    gb_ref[...] = hb_ref[...] * dis


def _tc1(ha, hb, dega, degb):
    return pl.pallas_call(
        _tc1_body,
        grid=(_GRID,),
        in_specs=[
            pl.BlockSpec((_BR, 128), lambda i: (i, 0)),
            pl.BlockSpec((_BR, 128), lambda i: (i, 0)),
            pl.BlockSpec((_BR, 16), lambda i: (i, 0)),
            pl.BlockSpec((_BR, 16), lambda i: (i, 0)),
        ],
        out_specs=[
            pl.BlockSpec((_BR, 128), lambda i: (i, 0)),
            pl.BlockSpec((_BR, 128), lambda i: (i, 0)),
        ],
        out_shape=[
            jax.ShapeDtypeStruct((N, 128), jnp.float32),
            jax.ShapeDtypeStruct((N, 128), jnp.float32),
        ],
    )(ha, hb, dega, degb)


def _tc2_body(agga_ref, aggb_ref, ga_ref, gb_ref, dega_ref, degb_ref,
              b1_ref, w2_ref, g2_ref):
    dis = _dis(dega_ref, degb_ref)[:, None]
    z0 = dis * (agga_ref[...] + ga_ref[...]) + b1_ref[0, :128][None, :]
    z1 = dis * (aggb_ref[...] + gb_ref[...]) + b1_ref[0, 128:][None, :]
    r0 = jnp.maximum(z0, 0.0)
    r1 = jnp.maximum(z1, 0.0)
    h2 = (lax.dot_general(r0, w2_ref[:128, :], (((1,), (0,)), ((), ())),
                          preferred_element_type=jnp.float32)
          + lax.dot_general(r1, w2_ref[128:, :], (((1,), (0,)), ((), ())),
                            preferred_element_type=jnp.float32))
    g2_ref[...] = h2 * dis


def _tc2(agg1a, agg1b, g1a, g1b, dega, degb, b1, W2):
    return pl.pallas_call(
        _tc2_body,
        grid=(_GRID,),
        in_specs=[
            pl.BlockSpec((_BR, 128), lambda i: (i, 0)),
            pl.BlockSpec((_BR, 128), lambda i: (i, 0)),
            pl.BlockSpec((_BR, 128), lambda i: (i, 0)),
            pl.BlockSpec((_BR, 128), lambda i: (i, 0)),
            pl.BlockSpec((_BR, 16), lambda i: (i, 0)),
            pl.BlockSpec((_BR, 16), lambda i: (i, 0)),
            pl.BlockSpec((1, D_HID), lambda i: (0, 0)),
            pl.BlockSpec((D_HID, D_OUT), lambda i: (0, 0)),
        ],
        out_specs=pl.BlockSpec((_BR, D_OUT), lambda i: (i, 0)),
        out_shape=jax.ShapeDtypeStruct((N, D_OUT), jnp.float32),
    )(agg1a, agg1b, g1a, g1b, dega, degb, b1, W2)


def _tc3_body(agga_ref, aggb_ref, g2_ref, dega_ref, degb_ref, b2_ref, out_ref):
    dis = _dis(dega_ref, degb_ref)[:, None]
    z = dis * (agga_ref[...] + aggb_ref[...] + g2_ref[...]) + b2_ref[0][None, :]
    out_ref[...] = jnp.maximum(z, 0.0)


def _tc3(agg2a, agg2b, g2, dega, degb, b2):
    return pl.pallas_call(
        _tc3_body,
        grid=(_GRID,),
        in_specs=[
            pl.BlockSpec((_BR, 128), lambda i: (i, 0)),
            pl.BlockSpec((_BR, 128), lambda i: (i, 0)),
            pl.BlockSpec((_BR, 128), lambda i: (i, 0)),
            pl.BlockSpec((_BR, 16), lambda i: (i, 0)),
            pl.BlockSpec((_BR, 16), lambda i: (i, 0)),
            pl.BlockSpec((1, D_OUT), lambda i: (0, 0)),
        ],
        out_specs=pl.BlockSpec((_BR, D_OUT), lambda i: (i, 0)),
        out_shape=jax.ShapeDtypeStruct((N, D_OUT), jnp.float32),
    )(agg2a, agg2b, g2, dega, degb, b2)


def _pack_eidx(src, dst, lead, groups):
    s4 = src.reshape(lead, groups, GC, KE)
    d4 = dst.reshape(lead, groups, GC, KE)
    return jnp.concatenate([s4, d4], axis=2)


def kernel(x, edge_index, W1, b1, W2, b2):
    ei = edge_index.astype(jnp.int32)
    src, dst = ei[0], ei[1]
    eidx1 = _pack_eidx(src, dst, NS, L1_GROUPS)
    eidx2 = _pack_eidx(src, dst, NC * NS, L2_GROUPS)
    dstD = dst.reshape(NC * NS, DEG_CHUNKS, KE)

    h1a, h1b = _tc0(x, W1)
    dega, degb = _make_deg_kernel()(dstD)
    g1a, g1b = _tc1(h1a, h1b, dega, degb)
    agg1a, agg1b = _make_agg1_kernel()(g1a, g1b, eidx1)
    g2 = _tc2(agg1a, agg1b, g1a, g1b, dega, degb, b1.reshape(1, D_HID), W2)
    agg2a, agg2b = _make_agg2_kernel()(g2, eidx2)
    return _tc3(agg2a, agg2b, g2, dega, degb, b2.reshape(1, D_OUT))


# speedup vs baseline: 28.1553x; 1.0023x over previous
import functools

import jax
import jax.numpy as jnp
from jax import lax
from jax.experimental import pallas as pl
from jax.experimental.pallas import tpu as pltpu
from jax.experimental.pallas import tpu_sc as plsc

N = 10000
E = 320000
D_IN = 128
D_HID = 256
D_OUT = 128

NC = 2
NS = 16

KE = 125
GC = 8
L1_GROUPS = E // NS // (GC * KE)
L2_GROUPS = E // (NC * NS) // (GC * KE)
DEG_CHUNKS = E // (NC * NS) // KE


def _zero_fill(buf, nrows, ncols):
    z16 = jnp.zeros((16,), jnp.float32)
    per_row = ncols // 16

    def body(t, carry):
        i = t // per_row
        k = t - i * per_row
        buf[i, pl.ds(k * 16, 16)] = z16
        return carry

    lax.fori_loop(0, nrows * per_row, body, 0)


def _zero_acc_slice(acc, zbuf, s, width):
    _zero_fill(zbuf, KE, width)
    for q in range(5):
        pltpu.sync_copy(zbuf, acc.at[pl.ds(s * 625 + q * KE, KE)])


def _agg_pipeline(tbl, eidx, lead, acc, ib, rb, semi, semg, sems, ngroups):
    nchunks = ngroups * GC

    pltpu.sync_copy(eidx.at[lead, 0], ib[0])
    pltpu.async_copy(tbl.at[ib[0].at[0]], rb[0], semg[0])

    def group_pair(gp, carry):
        for half in (0, 1):
            g = 2 * gp + half
            ibc, ibn = ib[half], ib[1 - half]
            for k in range(GC):
                j = g * GC + k
                p = k % 2
                rc, rn = rb[p], rb[1 - p]
                if k == 0:
                    @pl.when(g + 1 < ngroups)
                    def _():
                        pltpu.async_copy(eidx.at[lead, g + 1], ibn, semi)

                @pl.when(j > 0)
                def _():
                    pltpu.make_async_copy(rn, acc.at[ibc.at[GC]], sems[1 - p]).wait()

                if k < GC - 1:
                    pltpu.async_copy(tbl.at[ibc.at[k + 1]], rn, semg[1 - p])
                else:
                    @pl.when(j + 1 < nchunks)
                    def _():
                        pltpu.make_async_copy(eidx.at[lead, g + 1], ibn, semi).wait()
                        pltpu.async_copy(tbl.at[ibn.at[0]], rn, semg[1 - p])
                pltpu.make_async_copy(tbl.at[ibc.at[k]], rc, semg[p]).wait()
                pltpu.async_copy(rc, acc.at[ibc.at[GC + k]], sems[p], add=True)
        return carry

    lax.fori_loop(0, ngroups // 2, group_pair, 0)
    pltpu.make_async_copy(rb[1], acc.at[ib[1].at[GC]], sems[1]).wait()


def _flush(acc, out_h, c, s, core):
    @pl.when((c == core) & (s < 10))
    def _():
        sl = pl.ds(s * 1000, 1000)
        pltpu.sync_copy(acc.at[sl], out_h.at[sl])


def _sc_mesh():
    return plsc.VectorSubcoreMesh(core_axis_name="c", subcore_axis_name="s",
                                  num_cores=NC, num_subcores=NS)


@functools.cache
def _make_deg_kernel():
    return pl.kernel(
        _deg_body,
        out_type=(
            jax.ShapeDtypeStruct((N, 16), jnp.float32),
            jax.ShapeDtypeStruct((N, 16), jnp.float32),
        ),
        mesh=_sc_mesh(),
        scratch_types=(
            pltpu.VMEM_SHARED((N, 16), jnp.float32),
            pltpu.VMEM((DEG_CHUNKS, KE), jnp.int32),
            pltpu.VMEM((KE, 16), jnp.float32),
            pltpu.VMEM((KE, 16), jnp.float32),
            pltpu.SemaphoreType.DMA,
        ),
    )


def _deg_body(dst_h, dega_h, degb_h, acc, dst_v, ones_v, zbuf, semd):
    c = lax.axis_index("c")
    s = lax.axis_index("s")
    w = c * NS + s

    _zero_fill(zbuf, KE, 16)
    for q in range(5):
        pltpu.sync_copy(zbuf, acc.at[pl.ds(s * 625 + q * KE, KE)])

    one16 = jnp.ones((16,), jnp.float32)

    def fill_ones(i, carry):
        ones_v[i] = one16
        return carry

    lax.fori_loop(0, KE, fill_ones, 0)

    pltpu.sync_copy(dst_h.at[w], dst_v)
    plsc.subcore_barrier()

    def body(b, carry):
        for k in range(8):
            pltpu.async_copy(ones_v, acc.at[dst_v.at[8 * b + k]], semd, add=True)
        for k in range(8):
            pltpu.make_async_copy(ones_v, acc.at[dst_v.at[8 * b + k]], semd).wait()
        return carry

    lax.fori_loop(0, DEG_CHUNKS // 8, body, 0)
    plsc.subcore_barrier()

    _flush(acc, dega_h, c, s, 0)
    _flush(acc, degb_h, c, s, 1)


@functools.cache
def _make_agg1_kernel():
    return pl.kernel(
        _agg1_body,
        out_type=(
            jax.ShapeDtypeStruct((N, 128), jnp.float32),
            jax.ShapeDtypeStruct((N, 128), jnp.float32),
        ),
        mesh=_sc_mesh(),
        scratch_types=(
            pltpu.VMEM_SHARED((N, 128), jnp.float32),
            pltpu.VMEM((2 * GC, KE), jnp.int32),
            pltpu.VMEM((2 * GC, KE), jnp.int32),
            pltpu.VMEM((KE, 128), jnp.float32),
            pltpu.VMEM((KE, 128), jnp.float32),
            pltpu.SemaphoreType.DMA,
            pltpu.SemaphoreType.DMA,
            pltpu.SemaphoreType.DMA,
            pltpu.SemaphoreType.DMA,
            pltpu.SemaphoreType.DMA,
        ),
    )


def _agg1_body(ga_h, gb_h, eidx_h, outa_h, outb_h,
               acc, ib0, ib1, rb0, rb1, semi, semg0, semg1, sems0, sems1):
    c = lax.axis_index("c")
    s = lax.axis_index("s")

    _zero_acc_slice(acc, rb0, s, 128)
    plsc.subcore_barrier()

    @pl.when(c == 0)
    def _():
        _agg_pipeline(ga_h, eidx_h, s, acc, (ib0, ib1), (rb0, rb1),
                      semi, (semg0, semg1), (sems0, sems1), L1_GROUPS)

    @pl.when(c == 1)
    def _():
        _agg_pipeline(gb_h, eidx_h, s, acc, (ib0, ib1), (rb0, rb1),
                      semi, (semg0, semg1), (sems0, sems1), L1_GROUPS)

    plsc.subcore_barrier()
    _flush(acc, outa_h, c, s, 0)
    _flush(acc, outb_h, c, s, 1)


@functools.cache
def _make_agg2_kernel():
    return pl.kernel(
        _agg2_body,
        out_type=(
            jax.ShapeDtypeStruct((N, 128), jnp.float32),
            jax.ShapeDtypeStruct((N, 128), jnp.float32),
        ),
        mesh=_sc_mesh(),
        scratch_types=(
            pltpu.VMEM_SHARED((N, 128), jnp.float32),
            pltpu.VMEM((2 * GC, KE), jnp.int32),
            pltpu.VMEM((2 * GC, KE), jnp.int32),
            pltpu.VMEM((KE, 128), jnp.float32),
            pltpu.VMEM((KE, 128), jnp.float32),
            pltpu.SemaphoreType.DMA,
            pltpu.SemaphoreType.DMA,
            pltpu.SemaphoreType.DMA,
            pltpu.SemaphoreType.DMA,
            pltpu.SemaphoreType.DMA,
        ),
    )


def _agg2_body(g_h, eidx_h, outa_h, outb_h,
               acc, ib0, ib1, rb0, rb1, semi, semg0, semg1, sems0, sems1):
    c = lax.axis_index("c")
    s = lax.axis_index("s")
    w = c * NS + s

    _zero_acc_slice(acc, rb0, s, 128)
    plsc.subcore_barrier()

    _agg_pipeline(g_h, eidx_h, w, acc, (ib0, ib1), (rb0, rb1),
                  semi, (semg0, semg1), (sems0, sems1), L2_GROUPS)

    plsc.subcore_barrier()
    _flush(acc, outa_h, c, s, 0)
    _flush(acc, outb_h, c, s, 1)


_BR = 1000
_GRID = N // _BR


def _tc1_body(x_ref, w1_ref, dega_ref, degb_ref, ga_ref, gb_ref, dis_ref):
    h = lax.dot_general(x_ref[...], w1_ref[...], (((1,), (0,)), ((), ())),
                        preferred_element_type=jnp.float32)
    dis16 = lax.rsqrt(dega_ref[...] + degb_ref[...] + 1.0)
    dis_ref[...] = dis16
    dis = dis16[:, 0][:, None]
    g = h * dis
    ga_ref[...] = g[:, :128]
    gb_ref[...] = g[:, 128:]


def _tc1(x, W1, dega, degb):
    return pl.pallas_call(
        _tc1_body,
        grid=(_GRID,),
        in_specs=[
            pl.BlockSpec((_BR, D_IN), lambda i: (i, 0)),
            pl.BlockSpec((D_IN, D_HID), lambda i: (0, 0)),
            pl.BlockSpec((_BR, 16), lambda i: (i, 0)),
            pl.BlockSpec((_BR, 16), lambda i: (i, 0)),
        ],
        out_specs=[
            pl.BlockSpec((_BR, 128), lambda i: (i, 0)),
            pl.BlockSpec((_BR, 128), lambda i: (i, 0)),
            pl.BlockSpec((_BR, 16), lambda i: (i, 0)),
        ],
        out_shape=[
            jax.ShapeDtypeStruct((N, 128), jnp.float32),
            jax.ShapeDtypeStruct((N, 128), jnp.float32),
            jax.ShapeDtypeStruct((N, 16), jnp.float32),
        ],
    )(x, W1, dega, degb)


def _tc2_body(agga_ref, aggb_ref, ga_ref, gb_ref, dis_ref,
              b1_ref, w2_ref, g2_ref):
    dis = dis_ref[:, 0][:, None]
    z0 = dis * (agga_ref[...] + ga_ref[...]) + b1_ref[0, :128][None, :]
    z1 = dis * (aggb_ref[...] + gb_ref[...]) + b1_ref[0, 128:][None, :]
    r0 = jnp.maximum(z0, 0.0)
    r1 = jnp.maximum(z1, 0.0)
    h2 = (lax.dot_general(r0, w2_ref[:128, :], (((1,), (0,)), ((), ())),
                          preferred_element_type=jnp.float32)
          + lax.dot_general(r1, w2_ref[128:, :], (((1,), (0,)), ((), ())),
                            preferred_element_type=jnp.float32))
    g2_ref[...] = h2 * dis


def _tc2(agg1a, agg1b, g1a, g1b, dis16, b1, W2):
    return pl.pallas_call(
        _tc2_body,
        grid=(_GRID,),
        in_specs=[
            pl.BlockSpec((_BR, 128), lambda i: (i, 0)),
            pl.BlockSpec((_BR, 128), lambda i: (i, 0)),
            pl.BlockSpec((_BR, 128), lambda i: (i, 0)),
            pl.BlockSpec((_BR, 128), lambda i: (i, 0)),
            pl.BlockSpec((_BR, 16), lambda i: (i, 0)),
            pl.BlockSpec((1, D_HID), lambda i: (0, 0)),
            pl.BlockSpec((D_HID, D_OUT), lambda i: (0, 0)),
        ],
        out_specs=pl.BlockSpec((_BR, D_OUT), lambda i: (i, 0)),
        out_shape=jax.ShapeDtypeStruct((N, D_OUT), jnp.float32),
    )(agg1a, agg1b, g1a, g1b, dis16, b1, W2)


def _tc3_body(agga_ref, aggb_ref, g2_ref, dis_ref, b2_ref, out_ref):
    dis = dis_ref[:, 0][:, None]
    z = dis * (agga_ref[...] + aggb_ref[...] + g2_ref[...]) + b2_ref[0][None, :]
    out_ref[...] = jnp.maximum(z, 0.0)


def _tc3(agg2a, agg2b, g2, dis16, b2):
    return pl.pallas_call(
        _tc3_body,
        grid=(_GRID,),
        in_specs=[
            pl.BlockSpec((_BR, 128), lambda i: (i, 0)),
            pl.BlockSpec((_BR, 128), lambda i: (i, 0)),
            pl.BlockSpec((_BR, 128), lambda i: (i, 0)),
            pl.BlockSpec((_BR, 16), lambda i: (i, 0)),
            pl.BlockSpec((1, D_OUT), lambda i: (0, 0)),
        ],
        out_specs=pl.BlockSpec((_BR, D_OUT), lambda i: (i, 0)),
        out_shape=jax.ShapeDtypeStruct((N, D_OUT), jnp.float32),
    )(agg2a, agg2b, g2, dis16, b2)


def _pack_eidx(src, dst, lead, groups):
    s4 = src.reshape(lead, groups, GC, KE)
    d4 = dst.reshape(lead, groups, GC, KE)
    return jnp.concatenate([s4, d4], axis=2)


def kernel(x, edge_index, W1, b1, W2, b2):
    ei = edge_index.astype(jnp.int32)
    src, dst = ei[0], ei[1]
    eidx1 = _pack_eidx(src, dst, NS, L1_GROUPS)
    eidx2 = _pack_eidx(src, dst, NC * NS, L2_GROUPS)
    dstD = dst.reshape(NC * NS, DEG_CHUNKS, KE)

    dega, degb = _make_deg_kernel()(dstD)
    g1a, g1b, dis16 = _tc1(x, W1, dega, degb)
    agg1a, agg1b = _make_agg1_kernel()(g1a, g1b, eidx1)
    g2 = _tc2(agg1a, agg1b, g1a, g1b, dis16, b1.reshape(1, D_HID), W2)
    agg2a, agg2b = _make_agg2_kernel()(g2, eidx2)
    return _tc3(agg2a, agg2b, g2, dis16, b2.reshape(1, D_OUT))


# speedup vs baseline: 28.4076x; 1.0090x over previous
import functools

import jax
import jax.numpy as jnp
from jax import lax
from jax.experimental import pallas as pl
from jax.experimental.pallas import tpu as pltpu
from jax.experimental.pallas import tpu_sc as plsc

N = 10000
E = 320000
D_IN = 128
D_HID = 256
D_OUT = 128

NC = 2
NS = 16

KE = 125
GC = 8
L1_GROUPS = E // NS // (GC * KE)
L2_GROUPS = E // (NC * NS) // (GC * KE)
DEG_CHUNKS = E // (NC * NS) // KE


def _zero_fill(buf, nrows, ncols):
    z16 = jnp.zeros((16,), jnp.float32)
    per_row = ncols // 16

    def body(t, carry):
        i = t // per_row
        k = t - i * per_row
        buf[i, pl.ds(k * 16, 16)] = z16
        return carry

    lax.fori_loop(0, nrows * per_row, body, 0)


def _zero_acc_slice(acc, zbuf, s, width):
    _zero_fill(zbuf, KE, width)
    for q in range(5):
        pltpu.sync_copy(zbuf, acc.at[pl.ds(s * 625 + q * KE, KE)])


def _agg_pipeline(tbl, src4, dst4, lead, acc, ibs, ibd, rb,
                  semi, semg, sems, ngroups):
    nchunks = ngroups * GC

    pltpu.sync_copy(src4.at[lead, 0], ibs[0])
    pltpu.sync_copy(dst4.at[lead, 0], ibd[0])
    pltpu.async_copy(tbl.at[ibs[0].at[0]], rb[0], semg[0])

    def group_pair(gp, carry):
        for half in (0, 1):
            g = 2 * gp + half
            isc, isn = ibs[half], ibs[1 - half]
            idc, idn = ibd[half], ibd[1 - half]
            for k in range(GC):
                j = g * GC + k
                p = k % 2
                rc, rn = rb[p], rb[1 - p]
                if k == 0:
                    @pl.when(g + 1 < ngroups)
                    def _():
                        pltpu.async_copy(src4.at[lead, g + 1], isn, semi)
                        pltpu.async_copy(dst4.at[lead, g + 1], idn, semi)

                @pl.when(j > 0)
                def _():
                    pltpu.make_async_copy(rn, acc.at[idc.at[0]], sems[1 - p]).wait()

                if k < GC - 1:
                    pltpu.async_copy(tbl.at[isc.at[k + 1]], rn, semg[1 - p])
                else:
                    @pl.when(j + 1 < nchunks)
                    def _():
                        pltpu.make_async_copy(src4.at[lead, g + 1], isn, semi).wait()
                        pltpu.make_async_copy(dst4.at[lead, g + 1], idn, semi).wait()
                        pltpu.async_copy(tbl.at[isn.at[0]], rn, semg[1 - p])
                pltpu.make_async_copy(tbl.at[isc.at[k]], rc, semg[p]).wait()
                pltpu.async_copy(rc, acc.at[idc.at[k]], sems[p], add=True)
        return carry

    lax.fori_loop(0, ngroups // 2, group_pair, 0)
    pltpu.make_async_copy(rb[1], acc.at[ibd[1].at[0]], sems[1]).wait()


def _flush(acc, out_h, c, s, core):
    @pl.when((c == core) & (s < 10))
    def _():
        sl = pl.ds(s * 1000, 1000)
        pltpu.sync_copy(acc.at[sl], out_h.at[sl])


def _sc_mesh():
    return plsc.VectorSubcoreMesh(core_axis_name="c", subcore_axis_name="s",
                                  num_cores=NC, num_subcores=NS)


@functools.cache
def _make_deg_kernel():
    return pl.kernel(
        _deg_body,
        out_type=(
            jax.ShapeDtypeStruct((N, 16), jnp.float32),
            jax.ShapeDtypeStruct((N, 16), jnp.float32),
        ),
        mesh=_sc_mesh(),
        scratch_types=(
            pltpu.VMEM_SHARED((N, 16), jnp.float32),
            pltpu.VMEM((DEG_CHUNKS, KE), jnp.int32),
            pltpu.VMEM((KE, 16), jnp.float32),
            pltpu.VMEM((KE, 16), jnp.float32),
            pltpu.SemaphoreType.DMA,
        ),
    )


def _deg_body(dst_h, dega_h, degb_h, acc, dst_v, ones_v, zbuf, semd):
    c = lax.axis_index("c")
    s = lax.axis_index("s")
    w = c * NS + s

    _zero_fill(zbuf, KE, 16)
    for q in range(5):
        pltpu.sync_copy(zbuf, acc.at[pl.ds(s * 625 + q * KE, KE)])

    one16 = jnp.ones((16,), jnp.float32)

    def fill_ones(i, carry):
        ones_v[i] = one16
        return carry

    lax.fori_loop(0, KE, fill_ones, 0)

    pltpu.sync_copy(dst_h.at[w], dst_v)
    plsc.subcore_barrier()

    def body(b, carry):
        for k in range(16):
            pltpu.async_copy(ones_v, acc.at[dst_v.at[16 * b + k]], semd, add=True)
        for k in range(16):
            pltpu.make_async_copy(ones_v, acc.at[dst_v.at[16 * b + k]], semd).wait()
        return carry

    lax.fori_loop(0, DEG_CHUNKS // 16, body, 0)
    plsc.subcore_barrier()

    _flush(acc, dega_h, c, s, 0)
    _flush(acc, degb_h, c, s, 1)


@functools.cache
def _make_agg1_kernel():
    return pl.kernel(
        _agg1_body,
        out_type=(
            jax.ShapeDtypeStruct((N, 128), jnp.float32),
            jax.ShapeDtypeStruct((N, 128), jnp.float32),
        ),
        mesh=_sc_mesh(),
        scratch_types=(
            pltpu.VMEM_SHARED((N, 128), jnp.float32),
            pltpu.VMEM((GC, KE), jnp.int32),
            pltpu.VMEM((GC, KE), jnp.int32),
            pltpu.VMEM((GC, KE), jnp.int32),
            pltpu.VMEM((GC, KE), jnp.int32),
            pltpu.VMEM((KE, 128), jnp.float32),
            pltpu.VMEM((KE, 128), jnp.float32),
            pltpu.SemaphoreType.DMA,
            pltpu.SemaphoreType.DMA,
            pltpu.SemaphoreType.DMA,
            pltpu.SemaphoreType.DMA,
            pltpu.SemaphoreType.DMA,
        ),
    )


def _agg1_body(ga_h, gb_h, src_h, dst_h, outa_h, outb_h,
               acc, ibs0, ibs1, ibd0, ibd1, rb0, rb1,
               semi, semg0, semg1, sems0, sems1):
    c = lax.axis_index("c")
    s = lax.axis_index("s")

    _zero_acc_slice(acc, rb0, s, 128)
    plsc.subcore_barrier()

    @pl.when(c == 0)
    def _():
        _agg_pipeline(ga_h, src_h, dst_h, s, acc, (ibs0, ibs1), (ibd0, ibd1),
                      (rb0, rb1), semi, (semg0, semg1), (sems0, sems1), L1_GROUPS)

    @pl.when(c == 1)
    def _():
        _agg_pipeline(gb_h, src_h, dst_h, s, acc, (ibs0, ibs1), (ibd0, ibd1),
                      (rb0, rb1), semi, (semg0, semg1), (sems0, sems1), L1_GROUPS)

    plsc.subcore_barrier()
    _flush(acc, outa_h, c, s, 0)
    _flush(acc, outb_h, c, s, 1)


@functools.cache
def _make_agg2_kernel():
    return pl.kernel(
        _agg2_body,
        out_type=(
            jax.ShapeDtypeStruct((N, 128), jnp.float32),
            jax.ShapeDtypeStruct((N, 128), jnp.float32),
        ),
        mesh=_sc_mesh(),
        scratch_types=(
            pltpu.VMEM_SHARED((N, 128), jnp.float32),
            pltpu.VMEM((GC, KE), jnp.int32),
            pltpu.VMEM((GC, KE), jnp.int32),
            pltpu.VMEM((GC, KE), jnp.int32),
            pltpu.VMEM((GC, KE), jnp.int32),
            pltpu.VMEM((KE, 128), jnp.float32),
            pltpu.VMEM((KE, 128), jnp.float32),
            pltpu.SemaphoreType.DMA,
            pltpu.SemaphoreType.DMA,
            pltpu.SemaphoreType.DMA,
            pltpu.SemaphoreType.DMA,
            pltpu.SemaphoreType.DMA,
        ),
    )


def _agg2_body(g_h, src_h, dst_h, outa_h, outb_h,
               acc, ibs0, ibs1, ibd0, ibd1, rb0, rb1,
               semi, semg0, semg1, sems0, sems1):
    c = lax.axis_index("c")
    s = lax.axis_index("s")
    w = c * NS + s

    _zero_acc_slice(acc, rb0, s, 128)
    plsc.subcore_barrier()

    _agg_pipeline(g_h, src_h, dst_h, w, acc, (ibs0, ibs1), (ibd0, ibd1),
                  (rb0, rb1), semi, (semg0, semg1), (sems0, sems1), L2_GROUPS)

    plsc.subcore_barrier()
    _flush(acc, outa_h, c, s, 0)
    _flush(acc, outb_h, c, s, 1)


_BR = 2000
_GRID = N // _BR


def _tc1_body(x_ref, w1_ref, dega_ref, degb_ref, ga_ref, gb_ref, dis_ref):
    h = lax.dot_general(x_ref[...], w1_ref[...], (((1,), (0,)), ((), ())),
                        preferred_element_type=jnp.float32)
    dis16 = lax.rsqrt(dega_ref[...] + degb_ref[...] + 1.0)
    dis_ref[...] = dis16
    dis = dis16[:, 0][:, None]
    g = h * dis
    ga_ref[...] = g[:, :128]
    gb_ref[...] = g[:, 128:]


def _tc1(x, W1, dega, degb):
    return pl.pallas_call(
        _tc1_body,
        grid=(_GRID,),
        in_specs=[
            pl.BlockSpec((_BR, D_IN), lambda i: (i, 0)),
            pl.BlockSpec((D_IN, D_HID), lambda i: (0, 0)),
            pl.BlockSpec((_BR, 16), lambda i: (i, 0)),
            pl.BlockSpec((_BR, 16), lambda i: (i, 0)),
        ],
        out_specs=[
            pl.BlockSpec((_BR, 128), lambda i: (i, 0)),
            pl.BlockSpec((_BR, 128), lambda i: (i, 0)),
            pl.BlockSpec((_BR, 16), lambda i: (i, 0)),
        ],
        out_shape=[
            jax.ShapeDtypeStruct((N, 128), jnp.float32),
            jax.ShapeDtypeStruct((N, 128), jnp.float32),
            jax.ShapeDtypeStruct((N, 16), jnp.float32),
        ],
    )(x, W1, dega, degb)


def _tc2_body(agga_ref, aggb_ref, ga_ref, gb_ref, dis_ref,
              b1_ref, w2_ref, g2_ref):
    dis = dis_ref[:, 0][:, None]
    z0 = dis * (agga_ref[...] + ga_ref[...]) + b1_ref[0, :128][None, :]
    z1 = dis * (aggb_ref[...] + gb_ref[...]) + b1_ref[0, 128:][None, :]
    r0 = jnp.maximum(z0, 0.0)
    r1 = jnp.maximum(z1, 0.0)
    h2 = (lax.dot_general(r0, w2_ref[:128, :], (((1,), (0,)), ((), ())),
                          preferred_element_type=jnp.float32)
          + lax.dot_general(r1, w2_ref[128:, :], (((1,), (0,)), ((), ())),
                            preferred_element_type=jnp.float32))
    g2_ref[...] = h2 * dis


def _tc2(agg1a, agg1b, g1a, g1b, dis16, b1, W2):
    return pl.pallas_call(
        _tc2_body,
        grid=(_GRID,),
        in_specs=[
            pl.BlockSpec((_BR, 128), lambda i: (i, 0)),
            pl.BlockSpec((_BR, 128), lambda i: (i, 0)),
            pl.BlockSpec((_BR, 128), lambda i: (i, 0)),
            pl.BlockSpec((_BR, 128), lambda i: (i, 0)),
            pl.BlockSpec((_BR, 16), lambda i: (i, 0)),
            pl.BlockSpec((1, D_HID), lambda i: (0, 0)),
            pl.BlockSpec((D_HID, D_OUT), lambda i: (0, 0)),
        ],
        out_specs=pl.BlockSpec((_BR, D_OUT), lambda i: (i, 0)),
        out_shape=jax.ShapeDtypeStruct((N, D_OUT), jnp.float32),
    )(agg1a, agg1b, g1a, g1b, dis16, b1, W2)


def _tc3_body(agga_ref, aggb_ref, g2_ref, dis_ref, b2_ref, out_ref):
    dis = dis_ref[:, 0][:, None]
    z = dis * (agga_ref[...] + aggb_ref[...] + g2_ref[...]) + b2_ref[0][None, :]
    out_ref[...] = jnp.maximum(z, 0.0)


def _tc3(agg2a, agg2b, g2, dis16, b2):
    return pl.pallas_call(
        _tc3_body,
        grid=(_GRID,),
        in_specs=[
            pl.BlockSpec((_BR, 128), lambda i: (i, 0)),
            pl.BlockSpec((_BR, 128), lambda i: (i, 0)),
            pl.BlockSpec((_BR, 128), lambda i: (i, 0)),
            pl.BlockSpec((_BR, 16), lambda i: (i, 0)),
            pl.BlockSpec((1, D_OUT), lambda i: (0, 0)),
        ],
        out_specs=pl.BlockSpec((_BR, D_OUT), lambda i: (i, 0)),
        out_shape=jax.ShapeDtypeStruct((N, D_OUT), jnp.float32),
    )(agg2a, agg2b, g2, dis16, b2)


def kernel(x, edge_index, W1, b1, W2, b2):
    ei = edge_index.astype(jnp.int32)
    src, dst = ei[0], ei[1]
    src1 = src.reshape(NS, L1_GROUPS, GC, KE)
    dst1 = dst.reshape(NS, L1_GROUPS, GC, KE)
    src2 = src.reshape(NC * NS, L2_GROUPS, GC, KE)
    dst2 = dst.reshape(NC * NS, L2_GROUPS, GC, KE)
    dstD = dst.reshape(NC * NS, DEG_CHUNKS, KE)

    dega, degb = _make_deg_kernel()(dstD)
    g1a, g1b, dis16 = _tc1(x, W1, dega, degb)
    agg1a, agg1b = _make_agg1_kernel()(g1a, g1b, src1, dst1)
    g2 = _tc2(agg1a, agg1b, g1a, g1b, dis16, b1.reshape(1, D_HID), W2)
    agg2a, agg2b = _make_agg2_kernel()(g2, src2, dst2)
    return _tc3(agg2a, agg2b, g2, dis16, b2.reshape(1, D_OUT))


# speedup vs baseline: 28.5588x; 1.0053x over previous
import functools

import jax
import jax.numpy as jnp
from jax import lax
from jax.experimental import pallas as pl
from jax.experimental.pallas import tpu as pltpu
from jax.experimental.pallas import tpu_sc as plsc

N = 10000
E = 320000
D_IN = 128
D_HID = 256
D_OUT = 128

NC = 2
NS = 16

KE = 125
GC = 8
L1_GROUPS = E // NS // (GC * KE)
L2_GROUPS = E // (NC * NS) // (GC * KE)
DEG_CHUNKS = E // (NC * NS) // KE


def _zero_fill(buf, nrows, ncols):
    z16 = jnp.zeros((16,), jnp.float32)
    per_row = ncols // 16

    def body(t, carry):
        i = t // per_row
        k = t - i * per_row
        buf[i, pl.ds(k * 16, 16)] = z16
        return carry

    lax.fori_loop(0, nrows * per_row, body, 0)


def _zero_acc_slice(acc, zbuf, s, width):
    _zero_fill(zbuf, KE, width)
    for q in range(5):
        pltpu.sync_copy(zbuf, acc.at[pl.ds(s * 625 + q * KE, KE)])


def _agg_pipeline(tbl, src4, dst4, lead, acc, ibs, ibd, rb,
                  semi, semg, sems, ngroups):
    nchunks = ngroups * GC

    pltpu.sync_copy(src4.at[lead, 0], ibs[0])
    pltpu.sync_copy(dst4.at[lead, 0], ibd[0])
    pltpu.async_copy(tbl.at[ibs[0].at[0]], rb[0], semg[0])

    def group_pair(gp, carry):
        for half in (0, 1):
            g = 2 * gp + half
            isc, isn = ibs[half], ibs[1 - half]
            idc, idn = ibd[half], ibd[1 - half]
            for k in range(GC):
                j = g * GC + k
                p = k % 2
                rc, rn = rb[p], rb[1 - p]
                if k == 0:
                    @pl.when(g + 1 < ngroups)
                    def _():
                        pltpu.async_copy(src4.at[lead, g + 1], isn, semi)
                        pltpu.async_copy(dst4.at[lead, g + 1], idn, semi)

                @pl.when(j > 0)
                def _():
                    pltpu.make_async_copy(rn, acc.at[idc.at[0]], sems[1 - p]).wait()

                if k < GC - 1:
                    pltpu.async_copy(tbl.at[isc.at[k + 1]], rn, semg[1 - p])
                else:
                    @pl.when(j + 1 < nchunks)
                    def _():
                        pltpu.make_async_copy(src4.at[lead, g + 1], isn, semi).wait()
                        pltpu.make_async_copy(dst4.at[lead, g + 1], idn, semi).wait()
                        pltpu.async_copy(tbl.at[isn.at[0]], rn, semg[1 - p])
                pltpu.make_async_copy(tbl.at[isc.at[k]], rc, semg[p]).wait()
                pltpu.async_copy(rc, acc.at[idc.at[k]], sems[p], add=True)
        return carry

    lax.fori_loop(0, ngroups // 2, group_pair, 0)
    pltpu.make_async_copy(rb[1], acc.at[ibd[1].at[0]], sems[1]).wait()


def _flush(acc, out_h, c, s, core):
    @pl.when((c == core) & (s < 10))
    def _():
        sl = pl.ds(s * 1000, 1000)
        pltpu.sync_copy(acc.at[sl], out_h.at[sl])


def _sc_mesh():
    return plsc.VectorSubcoreMesh(core_axis_name="c", subcore_axis_name="s",
                                  num_cores=NC, num_subcores=NS)


@functools.cache
def _make_deg_kernel():
    return pl.kernel(
        _deg_body,
        out_type=(
            jax.ShapeDtypeStruct((N, 16), jnp.float32),
            jax.ShapeDtypeStruct((N, 16), jnp.float32),
        ),
        mesh=_sc_mesh(),
        scratch_types=(
            pltpu.VMEM_SHARED((N, 16), jnp.float32),
            pltpu.VMEM((DEG_CHUNKS, KE), jnp.int32),
            pltpu.VMEM((KE, 16), jnp.float32),
            pltpu.VMEM((KE, 16), jnp.float32),
            pltpu.SemaphoreType.DMA,
        ),
    )


def _deg_body(dst_h, dega_h, degb_h, acc, dst_v, ones_v, zbuf, semd):
    c = lax.axis_index("c")
    s = lax.axis_index("s")
    w = c * NS + s

    _zero_fill(zbuf, KE, 16)
    for q in range(5):
        pltpu.sync_copy(zbuf, acc.at[pl.ds(s * 625 + q * KE, KE)])

    one16 = jnp.ones((16,), jnp.float32)

    def fill_ones(i, carry):
        ones_v[i] = one16
        return carry

    lax.fori_loop(0, KE, fill_ones, 0)

    pltpu.sync_copy(dst_h.at[w], dst_v)
    plsc.subcore_barrier()

    def body(b, carry):
        for k in range(16):
            pltpu.async_copy(ones_v, acc.at[dst_v.at[16 * b + k]], semd, add=True)
        for k in range(16):
            pltpu.make_async_copy(ones_v, acc.at[dst_v.at[16 * b + k]], semd).wait()
        return carry

    lax.fori_loop(0, DEG_CHUNKS // 16, body, 0)
    plsc.subcore_barrier()

    _flush(acc, dega_h, c, s, 0)
    _flush(acc, degb_h, c, s, 1)


@functools.cache
def _make_agg1_kernel():
    return pl.kernel(
        _agg1_body,
        out_type=(
            jax.ShapeDtypeStruct((N, 128), jnp.float32),
            jax.ShapeDtypeStruct((N, 128), jnp.float32),
        ),
        mesh=_sc_mesh(),
        scratch_types=(
            pltpu.VMEM_SHARED((N, 128), jnp.float32),
            pltpu.VMEM((GC, KE), jnp.int32),
            pltpu.VMEM((GC, KE), jnp.int32),
            pltpu.VMEM((GC, KE), jnp.int32),
            pltpu.VMEM((GC, KE), jnp.int32),
            pltpu.VMEM((KE, 128), jnp.float32),
            pltpu.VMEM((KE, 128), jnp.float32),
            pltpu.SemaphoreType.DMA,
            pltpu.SemaphoreType.DMA,
            pltpu.SemaphoreType.DMA,
            pltpu.SemaphoreType.DMA,
            pltpu.SemaphoreType.DMA,
        ),
    )


def _agg1_body(ga_h, gb_h, src_h, dst_h, outa_h, outb_h,
               acc, ibs0, ibs1, ibd0, ibd1, rb0, rb1,
               semi, semg0, semg1, sems0, sems1):
    c = lax.axis_index("c")
    s = lax.axis_index("s")

    _zero_acc_slice(acc, rb0, s, 128)
    plsc.subcore_barrier()

    @pl.when(c == 0)
    def _():
        _agg_pipeline(ga_h, src_h, dst_h, s, acc, (ibs0, ibs1), (ibd0, ibd1),
                      (rb0, rb1), semi, (semg0, semg1), (sems0, sems1), L1_GROUPS)

    @pl.when(c == 1)
    def _():
        _agg_pipeline(gb_h, src_h, dst_h, s, acc, (ibs0, ibs1), (ibd0, ibd1),
                      (rb0, rb1), semi, (semg0, semg1), (sems0, sems1), L1_GROUPS)

    plsc.subcore_barrier()
    _flush(acc, outa_h, c, s, 0)
    _flush(acc, outb_h, c, s, 1)


@functools.cache
def _make_agg2_kernel():
    return pl.kernel(
        _agg2_body,
        out_type=(
            jax.ShapeDtypeStruct((N, 128), jnp.float32),
            jax.ShapeDtypeStruct((N, 128), jnp.float32),
        ),
        mesh=_sc_mesh(),
        scratch_types=(
            pltpu.VMEM_SHARED((N, 128), jnp.float32),
            pltpu.VMEM((GC, KE), jnp.int32),
            pltpu.VMEM((GC, KE), jnp.int32),
            pltpu.VMEM((GC, KE), jnp.int32),
            pltpu.VMEM((GC, KE), jnp.int32),
            pltpu.VMEM((KE, 128), jnp.float32),
            pltpu.VMEM((KE, 128), jnp.float32),
            pltpu.SemaphoreType.DMA,
            pltpu.SemaphoreType.DMA,
            pltpu.SemaphoreType.DMA,
            pltpu.SemaphoreType.DMA,
            pltpu.SemaphoreType.DMA,
        ),
    )


def _agg2_body(g_h, src_h, dst_h, outa_h, outb_h,
               acc, ibs0, ibs1, ibd0, ibd1, rb0, rb1,
               semi, semg0, semg1, sems0, sems1):
    c = lax.axis_index("c")
    s = lax.axis_index("s")
    w = c * NS + s

    _zero_acc_slice(acc, rb0, s, 128)
    plsc.subcore_barrier()

    _agg_pipeline(g_h, src_h, dst_h, w, acc, (ibs0, ibs1), (ibd0, ibd1),
                  (rb0, rb1), semi, (semg0, semg1), (sems0, sems1), L2_GROUPS)

    plsc.subcore_barrier()
    _flush(acc, outa_h, c, s, 0)
    _flush(acc, outb_h, c, s, 1)


_BR = 2000
_GRID = N // _BR


def _tc1_body(x_ref, w1_ref, dega_ref, degb_ref, ga_ref, gb_ref, dis_ref):
    h = lax.dot_general(x_ref[...], w1_ref[...], (((1,), (0,)), ((), ())),
                        preferred_element_type=jnp.float32)
    dis16 = lax.rsqrt(dega_ref[...] + degb_ref[...] + 1.0)
    dis_ref[...] = dis16
    dis = dis16[:, 0][:, None]
    g = h * dis
    ga_ref[...] = g[:, :128]
    gb_ref[...] = g[:, 128:]


def _tc1(x, W1, dega, degb):
    return pl.pallas_call(
        _tc1_body,
        grid=(_GRID,),
        in_specs=[
            pl.BlockSpec((_BR, D_IN), lambda i: (i, 0)),
            pl.BlockSpec((D_IN, D_HID), lambda i: (0, 0)),
            pl.BlockSpec((_BR, 16), lambda i: (i, 0)),
            pl.BlockSpec((_BR, 16), lambda i: (i, 0)),
        ],
        out_specs=[
            pl.BlockSpec((_BR, 128), lambda i: (i, 0)),
            pl.BlockSpec((_BR, 128), lambda i: (i, 0)),
            pl.BlockSpec((_BR, 16), lambda i: (i, 0)),
        ],
        out_shape=[
            jax.ShapeDtypeStruct((N, 128), jnp.float32),
            jax.ShapeDtypeStruct((N, 128), jnp.float32),
            jax.ShapeDtypeStruct((N, 16), jnp.float32),
        ],
    )(x, W1, dega, degb)


def _tc2_body(agga_ref, aggb_ref, ga_ref, gb_ref, dis_ref,
              b1_ref, w2_ref, g2_ref):
    dis = dis_ref[:, 0][:, None]
    z0 = dis * (agga_ref[...] + ga_ref[...]) + b1_ref[0, :128][None, :]
    z1 = dis * (aggb_ref[...] + gb_ref[...]) + b1_ref[0, 128:][None, :]
    r0 = jnp.maximum(z0, 0.0)
    r1 = jnp.maximum(z1, 0.0)
    h2 = (lax.dot_general(r0, w2_ref[:128, :], (((1,), (0,)), ((), ())),
                          preferred_element_type=jnp.float32)
          + lax.dot_general(r1, w2_ref[128:, :], (((1,), (0,)), ((), ())),
                            preferred_element_type=jnp.float32))
    g2_ref[...] = h2 * dis


def _tc2(agg1a, agg1b, g1a, g1b, dis16, b1, W2):
    return pl.pallas_call(
        _tc2_body,
        grid=(_GRID,),
        in_specs=[
            pl.BlockSpec((_BR, 128), lambda i: (i, 0)),
            pl.BlockSpec((_BR, 128), lambda i: (i, 0)),
            pl.BlockSpec((_BR, 128), lambda i: (i, 0)),
            pl.BlockSpec((_BR, 128), lambda i: (i, 0)),
            pl.BlockSpec((_BR, 16), lambda i: (i, 0)),
            pl.BlockSpec((1, D_HID), lambda i: (0, 0)),
            pl.BlockSpec((D_HID, D_OUT), lambda i: (0, 0)),
        ],
        out_specs=pl.BlockSpec((_BR, D_OUT), lambda i: (i, 0)),
        out_shape=jax.ShapeDtypeStruct((N, D_OUT), jnp.float32),
    )(agg1a, agg1b, g1a, g1b, dis16, b1, W2)


def _tc3_body(agga_ref, aggb_ref, g2_ref, dis_ref, b2_ref, out_ref):
    dis = dis_ref[:, 0][:, None]
    z = dis * (agga_ref[...] + aggb_ref[...] + g2_ref[...]) + b2_ref[0][None, :]
    out_ref[...] = jnp.maximum(z, 0.0)


def _tc3(agg2a, agg2b, g2, dis16, b2):
    return pl.pallas_call(
        _tc3_body,
        grid=(_GRID,),
        in_specs=[
            pl.BlockSpec((_BR, 128), lambda i: (i, 0)),
            pl.BlockSpec((_BR, 128), lambda i: (i, 0)),
            pl.BlockSpec((_BR, 128), lambda i: (i, 0)),
            pl.BlockSpec((_BR, 16), lambda i: (i, 0)),
            pl.BlockSpec((1, D_OUT), lambda i: (0, 0)),
        ],
        out_specs=pl.BlockSpec((_BR, D_OUT), lambda i: (i, 0)),
        out_shape=jax.ShapeDtypeStruct((N, D_OUT), jnp.float32),
    )(agg2a, agg2b, g2, dis16, b2)


def kernel(x, edge_index, W1, b1, W2, b2):
    ei = edge_index.astype(jnp.int32)
    src, dst = ei[0], ei[1]
    dstD = dst.reshape(NC * NS, DEG_CHUNKS, KE)

    dega, degb = _make_deg_kernel()(dstD)

    src, dst = lax.optimization_barrier((src, dst))
    src1 = src.reshape(NS, L1_GROUPS, GC, KE)
    dst1 = dst.reshape(NS, L1_GROUPS, GC, KE)
    src2 = src.reshape(NC * NS, L2_GROUPS, GC, KE)
    dst2 = dst.reshape(NC * NS, L2_GROUPS, GC, KE)
    g1a, g1b, dis16 = _tc1(x, W1, dega, degb)
    agg1a, agg1b = _make_agg1_kernel()(g1a, g1b, src1, dst1)
    g2 = _tc2(agg1a, agg1b, g1a, g1b, dis16, b1.reshape(1, D_HID), W2)
    agg2a, agg2b = _make_agg2_kernel()(g2, src2, dst2)
    return _tc3(agg2a, agg2b, g2, dis16, b2.reshape(1, D_OUT))
